# baseline (speedup 1.0000x reference)
import functools

import jax
import jax.numpy as jnp
from jax import lax
from jax.experimental import pallas as pl
from jax.experimental.pallas import tpu as pltpu
from jax.experimental.pallas import tpu_sc as plsc

B, H, L1 = 1024, 200, 201
V, D, R = 1000000, 64, 6
S = L1 - 1
N = B * H

_NC = 2
_NS = 16
_NW = _NC * _NS
_CHUNK = 128
_PER_W = N // _NW
_NCH = _PER_W // _CHUNK


def _gather_body(ct, et, idx1, idx2, out1, out2, idxv, bufa, bufb, sema, semb):
    wid = lax.axis_index("s") * _NC + lax.axis_index("c")
    base = wid * _PER_W
    for tab, idx_hbm, out in ((ct, idx1, out1), (et, idx2, out2)):
        pltpu.sync_copy(idx_hbm.at[wid], idxv)

        def start(j, buf, sem, tab=tab):
            pltpu.make_async_copy(tab.at[idxv.at[j]], buf, sem).start()

        def wait(buf, sem, tab=tab):
            pltpu.make_async_copy(tab.at[idxv.at[0]], buf, sem).wait()

        def store(j, buf, out=out):
            pltpu.sync_copy(buf, out.at[pl.ds(base + j * _CHUNK, _CHUNK)])

        start(0, bufa, sema)
        start(1, bufb, semb)

        def body(t, carry, start=start, wait=wait, store=store):
            j = 2 * t
            wait(bufa, sema)
            store(j, bufa)

            @pl.when(j + 2 < _NCH)
            def _():
                start(j + 2, bufa, sema)

            wait(bufb, semb)
            store(j + 1, bufb)

            @pl.when(j + 3 < _NCH)
            def _():
                start(j + 3, bufb, semb)

            return carry

        lax.fori_loop(0, _NCH // 2, body, 0)


_gather = functools.partial(
    pl.kernel,
    mesh=plsc.VectorSubcoreMesh(core_axis_name="c", subcore_axis_name="s"),
    out_type=[
        jax.ShapeDtypeStruct((N, D), jnp.float32),
        jax.ShapeDtypeStruct((N, D), jnp.float32),
    ],
    scratch_types=[
        pltpu.VMEM((_NCH, _CHUNK), jnp.int32),
        pltpu.VMEM((_CHUNK, D), jnp.float32),
        pltpu.VMEM((_CHUNK, D), jnp.float32),
        pltpu.SemaphoreType.DMA,
        pltpu.SemaphoreType.DMA,
    ],
    compiler_params=pltpu.CompilerParams(use_tc_tiling_on_sc=False),
)(_gather_body)


_BLK = 4096


def _fused_body(g1, g2, bt, r1, st, ids, tsw, tsb, rtab, expw, expb, clkw, clkb,
                o1, o2):
    w_clk = clkw[...]
    w_exp = expw[...]
    tsw_v = tsw[...]
    tsb_v = tsb[...]
    rt = rtab[...]

    uc = jnp.dot(tsw_v, w_clk[D:2 * D], preferred_element_type=jnp.float32)
    ue = jnp.dot(tsw_v, w_exp[D:2 * D], preferred_element_type=jnp.float32)
    oh2 = (lax.broadcasted_iota(jnp.int32, (1, 8), 1) == 2).astype(jnp.float32)
    r2 = jnp.dot(oh2, rt, preferred_element_type=jnp.float32)
    cc = (jnp.dot(tsb_v, w_clk[D:2 * D], preferred_element_type=jnp.float32)
          + jnp.dot(r2, w_clk[2 * D:], preferred_element_type=jnp.float32)
          + clkb[...])
    ce = (jnp.dot(tsb_v, w_exp[D:2 * D], preferred_element_type=jnp.float32)
          + expb[...])
    rt6 = jnp.dot(rt, w_exp[2 * D:], preferred_element_type=jnp.float32)

    l1 = jnp.log(bt[...] - r1[...] + 1.0)
    l2 = jnp.log(bt[...] - st[...] + 1.0)
    oh = (ids[...] == lax.broadcasted_iota(jnp.int32, (_BLK, 8), 1)
          ).astype(jnp.float32)

    o1[...] = (jnp.dot(g1[...], w_clk[:D], preferred_element_type=jnp.float32)
               + l1 * uc + cc)
    o2[...] = (jnp.dot(g2[...], w_exp[:D], preferred_element_type=jnp.float32)
               + l2 * ue
               + jnp.dot(oh, rt6, preferred_element_type=jnp.float32)
               + ce)


_fused = pl.pallas_call(
    _fused_body,
    grid=(N // _BLK,),
    in_specs=[
        pl.BlockSpec((_BLK, D), lambda i: (i, 0)),
        pl.BlockSpec((_BLK, D), lambda i: (i, 0)),
        pl.BlockSpec((_BLK, 1), lambda i: (i, 0)),
        pl.BlockSpec((_BLK, 1), lambda i: (i, 0)),
        pl.BlockSpec((_BLK, 1), lambda i: (i, 0)),
        pl.BlockSpec((_BLK, 1), lambda i: (i, 0)),
        pl.BlockSpec((1, D), lambda i: (0, 0)),
        pl.BlockSpec((1, D), lambda i: (0, 0)),
        pl.BlockSpec((8, D), lambda i: (0, 0)),
        pl.BlockSpec((3 * D, D), lambda i: (0, 0)),
        pl.BlockSpec((1, D), lambda i: (0, 0)),
        pl.BlockSpec((3 * D, D), lambda i: (0, 0)),
        pl.BlockSpec((1, D), lambda i: (0, 0)),
    ],
    out_specs=[
        pl.BlockSpec((_BLK, D), lambda i: (i, 0)),
        pl.BlockSpec((_BLK, D), lambda i: (i, 0)),
    ],
    out_shape=[
        jax.ShapeDtypeStruct((N, D), jnp.float32),
        jax.ShapeDtypeStruct((N, D), jnp.float32),
    ],
)


def kernel(row0, row1, row2, row3, row4, row5, row6, row7, click_table,
           exposure_table, rating_table, ts_w, ts_b, exp_w, exp_b, clk_w,
           clk_b):
    item_time = row6[:, -1]
    seq_items = row4[:, :-1]
    seq_ratings = row5[:, :-1]
    seq_times = row6[:, :-1]

    idx1 = row0.astype(jnp.int32).reshape(_NW, _NCH, _CHUNK)
    idx2 = seq_items.astype(jnp.int32).reshape(_NW, _NCH, _CHUNK)
    g1, g2 = _gather(click_table, exposure_table, idx1, idx2)

    bt = jnp.broadcast_to(item_time[:, None], (B, H)).reshape(N, 1)
    r1f = row1.reshape(N, 1)
    stf = seq_times.reshape(N, 1)
    ids = seq_ratings.astype(jnp.int32).reshape(N, 1)
    rt8 = jnp.zeros((8, D), jnp.float32).at[:R].set(rating_table)
    o1, o2 = _fused(g1, g2, bt, r1f, stf, ids, ts_w, ts_b.reshape(1, D), rt8,
                    exp_w, exp_b.reshape(1, D), clk_w, clk_b.reshape(1, D))
    return o1.reshape(B, H, D), o2.reshape(B, S, D)


# speedup vs baseline: 1.3708x; 1.3708x over previous
import functools

import jax
import jax.numpy as jnp
from jax import lax
from jax.experimental import pallas as pl
from jax.experimental.pallas import tpu as pltpu
from jax.experimental.pallas import tpu_sc as plsc

B, H, L1 = 1024, 200, 201
V, D, R = 1000000, 64, 6
S = L1 - 1
N = B * H

_NC = 2
_NS = 16
_NW = _NC * _NS
_CHUNK = 128
_PER_W = 2 * N // _NW
_NCH = _PER_W // _CHUNK
_NBUF = 4


def _gather_body(pt, idx, out, idxv, bufs, sems):
    wid = lax.axis_index("s") * _NC + lax.axis_index("c")
    base = wid * _PER_W
    pltpu.sync_copy(idx.at[wid], idxv)

    def start(j, k):
        pltpu.make_async_copy(pt.at[idxv.at[j]], bufs[k], sems[k]).start()

    def wait(k):
        pltpu.make_async_copy(pt.at[idxv.at[0]], bufs[k], sems[k]).wait()

    def store(j, k):
        pltpu.sync_copy(bufs[k], out.at[pl.ds(base + j * _CHUNK, _CHUNK)])

    for k in range(_NBUF):
        start(k, k)

    def body(t, carry):
        j = t * _NBUF
        for k in range(_NBUF):
            wait(k)
            store(j + k, k)

            @pl.when(j + k + _NBUF < _NCH)
            def _():
                start(j + k + _NBUF, k)

        return carry

    lax.fori_loop(0, _NCH // _NBUF, body, 0)


def _gather_fn(pt, idx):
    scratch = [pltpu.VMEM((_NCH, _CHUNK), jnp.int32)]
    scratch += [pltpu.VMEM((_CHUNK, 2 * D), jnp.float32) for _ in range(_NBUF)]
    scratch += [pltpu.SemaphoreType.DMA for _ in range(_NBUF)]

    def body(pt_ref, idx_ref, out_ref, idxv, b0, b1, b2, b3, s0, s1, s2, s3):
        _gather_body(pt_ref, idx_ref, out_ref, idxv,
                     (b0, b1, b2, b3), (s0, s1, s2, s3))

    return pl.kernel(
        body,
        mesh=plsc.VectorSubcoreMesh(core_axis_name="c", subcore_axis_name="s"),
        out_type=jax.ShapeDtypeStruct((2 * N, 2 * D), jnp.float32),
        scratch_types=scratch,
        compiler_params=pltpu.CompilerParams(use_tc_tiling_on_sc=True),
    )(pt, idx)


_BB = 16
_BLK = _BB * S
_GRID = B // _BB


def _fused_body(g1, g2, it, r1, st, ids, tsw, tsb, rtab, expw, expb, clkw,
                clkb, o1, o2):
    w_clk = clkw[...]
    w_exp = expw[...]
    tsw_v = tsw[...]
    tsb_v = tsb[...]
    rt = rtab[...]
    zero = jnp.zeros((D, D), jnp.float32)
    wc1s = jnp.concatenate([w_clk[:D], zero], axis=0)
    we1s = jnp.concatenate([zero, w_exp[:D]], axis=0)

    uc = jnp.dot(tsw_v, w_clk[D:2 * D], preferred_element_type=jnp.float32)
    ue = jnp.dot(tsw_v, w_exp[D:2 * D], preferred_element_type=jnp.float32)
    oh2 = (lax.broadcasted_iota(jnp.int32, (1, 8), 1) == 2).astype(jnp.float32)
    r2 = jnp.dot(oh2, rt, preferred_element_type=jnp.float32)
    cc = (jnp.dot(tsb_v, w_clk[D:2 * D], preferred_element_type=jnp.float32)
          + jnp.dot(r2, w_clk[2 * D:], preferred_element_type=jnp.float32)
          + clkb[...])
    ce = (jnp.dot(tsb_v, w_exp[D:2 * D], preferred_element_type=jnp.float32)
          + expb[...])
    rt6 = jnp.dot(rt, w_exp[2 * D:], preferred_element_type=jnp.float32)

    itv = it[...]
    l1 = jnp.log(itv - r1[...] + 1.0)
    l2 = jnp.log(itv - st[...] + 1.0)
    L1 = jnp.broadcast_to(l1[:, :, None], (_BB, S, D)).reshape(_BLK, D)
    L2 = jnp.broadcast_to(l2[:, :, None], (_BB, S, D)).reshape(_BLK, D)
    oh = (ids[...][:, :, None]
          == lax.broadcasted_iota(jnp.int32, (_BB, S, 8), 2)
          ).astype(jnp.float32).reshape(_BLK, 8)

    out1 = (jnp.dot(g1[...], wc1s, preferred_element_type=jnp.float32)
            + L1 * uc + cc)
    out2 = (jnp.dot(g2[...], we1s, preferred_element_type=jnp.float32)
            + L2 * ue
            + jnp.dot(oh, rt6, preferred_element_type=jnp.float32)
            + ce)
    o1[...] = out1.reshape(_BB, S, D)
    o2[...] = out2.reshape(_BB, S, D)


_fused = pl.pallas_call(
    _fused_body,
    grid=(_GRID,),
    in_specs=[
        pl.BlockSpec((_BLK, 2 * D), lambda i: (i, 0)),
        pl.BlockSpec((_BLK, 2 * D), lambda i: (i + _GRID, 0)),
        pl.BlockSpec((_BB, 1), lambda i: (i, 0)),
        pl.BlockSpec((_BB, S), lambda i: (i, 0)),
        pl.BlockSpec((_BB, S), lambda i: (i, 0)),
        pl.BlockSpec((_BB, S), lambda i: (i, 0)),
        pl.BlockSpec((1, D), lambda i: (0, 0)),
        pl.BlockSpec((1, D), lambda i: (0, 0)),
        pl.BlockSpec((8, D), lambda i: (0, 0)),
        pl.BlockSpec((3 * D, D), lambda i: (0, 0)),
        pl.BlockSpec((1, D), lambda i: (0, 0)),
        pl.BlockSpec((3 * D, D), lambda i: (0, 0)),
        pl.BlockSpec((1, D), lambda i: (0, 0)),
    ],
    out_specs=[
        pl.BlockSpec((_BB, S, D), lambda i: (i, 0, 0)),
        pl.BlockSpec((_BB, S, D), lambda i: (i, 0, 0)),
    ],
    out_shape=[
        jax.ShapeDtypeStruct((B, H, D), jnp.float32),
        jax.ShapeDtypeStruct((B, S, D), jnp.float32),
    ],
)


def kernel(row0, row1, row2, row3, row4, row5, row6, row7, click_table,
           exposure_table, rating_table, ts_w, ts_b, exp_w, exp_b, clk_w,
           clk_b):
    item_time = row6[:, -1]
    seq_items = row4[:, :-1]
    seq_ratings = row5[:, :-1]
    seq_times = row6[:, :-1]

    pt = jnp.concatenate([click_table, exposure_table], axis=1)
    idx = jnp.concatenate(
        [row0.astype(jnp.int32).reshape(-1),
         seq_items.astype(jnp.int32).reshape(-1)]
    ).reshape(_NW, _NCH, _CHUNK)
    g = _gather_fn(pt, idx)

    rt8 = jnp.zeros((8, D), jnp.float32).at[:R].set(rating_table)
    o1, o2 = _fused(g, g, item_time.reshape(B, 1), row1, seq_times,
                    seq_ratings.astype(jnp.int32), ts_w, ts_b.reshape(1, D),
                    rt8, exp_w, exp_b.reshape(1, D), clk_w,
                    clk_b.reshape(1, D))
    return o1, o2


# speedup vs baseline: 1.9463x; 1.4198x over previous
import functools

import jax
import jax.numpy as jnp
from jax import lax
from jax.experimental import pallas as pl
from jax.experimental.pallas import tpu as pltpu
from jax.experimental.pallas import tpu_sc as plsc

B, H, L1 = 1024, 200, 201
V, D, R = 1000000, 64, 6
S = L1 - 1
N = B * H

_NC = 2
_NS = 16
_NW = _NC * _NS
_CHUNK = 128
_PER_W = 2 * N // _NW
_NCH = _PER_W // _CHUNK
_NBUF = 4


def _gather_body(pt, idx, out, idxv, bufs, sems):
    wid = lax.axis_index("s") * _NC + lax.axis_index("c")
    base = wid * _PER_W
    pltpu.sync_copy(idx.at[wid], idxv)

    def start(j, k):
        pltpu.make_async_copy(pt.at[idxv.at[j]], bufs[k], sems[k]).start()

    def wait(k):
        pltpu.make_async_copy(pt.at[idxv.at[0]], bufs[k], sems[k]).wait()

    def store(j, k):
        pltpu.sync_copy(bufs[k], out.at[pl.ds(base + j * _CHUNK, _CHUNK)])

    for k in range(_NBUF):
        start(k, k)

    def body(t, carry):
        j = t * _NBUF
        for k in range(_NBUF):
            wait(k)
            store(j + k, k)

            @pl.when(j + k + _NBUF < _NCH)
            def _():
                start(j + k + _NBUF, k)

        return carry

    lax.fori_loop(0, _NCH // _NBUF, body, 0)


def _gather_fn(pt, idx):
    scratch = [pltpu.VMEM((_NCH, _CHUNK), jnp.int32)]
    scratch += [pltpu.VMEM((_CHUNK, 2 * D), jnp.float32) for _ in range(_NBUF)]
    scratch += [pltpu.SemaphoreType.DMA for _ in range(_NBUF)]

    def body(pt_ref, idx_ref, out_ref, idxv, b0, b1, b2, b3, s0, s1, s2, s3):
        _gather_body(pt_ref, idx_ref, out_ref, idxv,
                     (b0, b1, b2, b3), (s0, s1, s2, s3))

    return pl.kernel(
        body,
        mesh=plsc.VectorSubcoreMesh(core_axis_name="c", subcore_axis_name="s"),
        out_type=jax.ShapeDtypeStruct((2 * N, 2 * D), jnp.float32),
        scratch_types=scratch,
        compiler_params=pltpu.CompilerParams(use_tc_tiling_on_sc=True),
    )(pt, idx)


_VB = 8192


def _premul_body(ctt, ett, clkw, expw, rtab, tsb, clkb, expb, out):
    w_clk = clkw[...]
    w_exp = expw[...]
    tsb_v = tsb[...]
    rt = rtab[...]
    oh2 = (lax.broadcasted_iota(jnp.int32, (1, 8), 1) == 2).astype(jnp.float32)
    r2 = jnp.dot(oh2, rt, preferred_element_type=jnp.float32)
    cc = (jnp.dot(tsb_v, w_clk[D:2 * D], preferred_element_type=jnp.float32)
          + jnp.dot(r2, w_clk[2 * D:], preferred_element_type=jnp.float32)
          + clkb[...])
    ce = (jnp.dot(tsb_v, w_exp[D:2 * D], preferred_element_type=jnp.float32)
          + expb[...])
    dn = (((0,), (0,)), ((), ()))
    pc = lax.dot_general(ctt[...], w_clk[:D], dn,
                         preferred_element_type=jnp.float32)
    pe = lax.dot_general(ett[...], w_exp[:D], dn,
                         preferred_element_type=jnp.float32)
    out[...] = jnp.concatenate([pc + cc, pe + ce], axis=1)


_premul = pl.pallas_call(
    _premul_body,
    grid=(pl.cdiv(V, _VB),),
    in_specs=[
        pl.BlockSpec((D, _VB), lambda i: (0, i)),
        pl.BlockSpec((D, _VB), lambda i: (0, i)),
        pl.BlockSpec((3 * D, D), lambda i: (0, 0)),
        pl.BlockSpec((3 * D, D), lambda i: (0, 0)),
        pl.BlockSpec((8, D), lambda i: (0, 0)),
        pl.BlockSpec((1, D), lambda i: (0, 0)),
        pl.BlockSpec((1, D), lambda i: (0, 0)),
        pl.BlockSpec((1, D), lambda i: (0, 0)),
    ],
    out_specs=pl.BlockSpec((_VB, 2 * D), lambda i: (i, 0)),
    out_shape=jax.ShapeDtypeStruct((V, 2 * D), jnp.float32),
)


_BB = 16
_BLK = _BB * S
_GRID = B // _BB


def _fused_body(g1, g2, it, r1, st, ids, tsw, rtab, expw, clkw, o1, o2):
    w_clk = clkw[...]
    w_exp = expw[...]
    tsw_v = tsw[...]
    rt = rtab[...]

    uc = jnp.dot(tsw_v, w_clk[D:2 * D], preferred_element_type=jnp.float32)
    ue = jnp.dot(tsw_v, w_exp[D:2 * D], preferred_element_type=jnp.float32)
    rt6 = jnp.dot(rt, w_exp[2 * D:], preferred_element_type=jnp.float32)

    itv = it[...]
    l1 = jnp.log(itv - r1[...] + 1.0)
    l2 = jnp.log(itv - st[...] + 1.0)
    L1 = jnp.broadcast_to(l1[:, :, None], (_BB, S, D)).reshape(_BLK, D)
    L2 = jnp.broadcast_to(l2[:, :, None], (_BB, S, D)).reshape(_BLK, D)
    oh = (ids[...][:, :, None]
          == lax.broadcasted_iota(jnp.int32, (_BB, S, 8), 2)
          ).astype(jnp.float32).reshape(_BLK, 8)

    out1 = g1[...][:, :D] + L1 * uc
    out2 = (g2[...][:, D:] + L2 * ue
            + jnp.dot(oh, rt6, preferred_element_type=jnp.float32))
    o1[...] = out1.reshape(_BB, S, D)
    o2[...] = out2.reshape(_BB, S, D)


_fused = pl.pallas_call(
    _fused_body,
    grid=(_GRID,),
    in_specs=[
        pl.BlockSpec((_BLK, 2 * D), lambda i: (i, 0)),
        pl.BlockSpec((_BLK, 2 * D), lambda i: (i + _GRID, 0)),
        pl.BlockSpec((_BB, 1), lambda i: (i, 0)),
        pl.BlockSpec((_BB, S), lambda i: (i, 0)),
        pl.BlockSpec((_BB, S), lambda i: (i, 0)),
        pl.BlockSpec((_BB, S), lambda i: (i, 0)),
        pl.BlockSpec((1, D), lambda i: (0, 0)),
        pl.BlockSpec((8, D), lambda i: (0, 0)),
        pl.BlockSpec((3 * D, D), lambda i: (0, 0)),
        pl.BlockSpec((3 * D, D), lambda i: (0, 0)),
    ],
    out_specs=[
        pl.BlockSpec((_BB, S, D), lambda i: (i, 0, 0)),
        pl.BlockSpec((_BB, S, D), lambda i: (i, 0, 0)),
    ],
    out_shape=[
        jax.ShapeDtypeStruct((B, H, D), jnp.float32),
        jax.ShapeDtypeStruct((B, S, D), jnp.float32),
    ],
)


def kernel(row0, row1, row2, row3, row4, row5, row6, row7, click_table,
           exposure_table, rating_table, ts_w, ts_b, exp_w, exp_b, clk_w,
           clk_b):
    item_time = row6[:, -1]
    seq_items = row4[:, :-1]
    seq_ratings = row5[:, :-1]
    seq_times = row6[:, :-1]

    rt8 = jnp.zeros((8, D), jnp.float32).at[:R].set(rating_table)
    pt = _premul(click_table.T, exposure_table.T, clk_w, exp_w, rt8,
                 ts_b.reshape(1, D), clk_b.reshape(1, D),
                 exp_b.reshape(1, D))
    idx = jnp.concatenate(
        [row0.astype(jnp.int32).reshape(-1),
         seq_items.astype(jnp.int32).reshape(-1)]
    ).reshape(_NW, _NCH, _CHUNK)
    g = _gather_fn(pt, idx)

    o1, o2 = _fused(g, g, item_time.reshape(B, 1), row1, seq_times,
                    seq_ratings.astype(jnp.int32), ts_w, rt8, exp_w, clk_w)
    return o1, o2


# speedup vs baseline: 2.1728x; 1.1164x over previous
import functools

import jax
import jax.numpy as jnp
from jax import lax
from jax.experimental import pallas as pl
from jax.experimental.pallas import tpu as pltpu
from jax.experimental.pallas import tpu_sc as plsc

B, H, L1 = 1024, 200, 201
V, D, R = 1000000, 64, 6
S = L1 - 1
N = B * H

_NC = 2
_NS = 16
_NW = _NC * _NS
_CHUNK = 128
_PER_W = 2 * N // _NW
_NCH = _PER_W // _CHUNK
_NBUF = 4


def _gather_body(pt, idx, out, idxv, bufs, sems):
    wid = lax.axis_index("s") * _NC + lax.axis_index("c")
    base = wid * _PER_W
    pltpu.sync_copy(idx.at[wid], idxv)

    def start(j, k):
        pltpu.make_async_copy(pt.at[idxv.at[j]], bufs[k], sems[k]).start()

    def wait(k):
        pltpu.make_async_copy(pt.at[idxv.at[0]], bufs[k], sems[k]).wait()

    def store(j, k):
        pltpu.sync_copy(bufs[k], out.at[pl.ds(base + j * _CHUNK, _CHUNK)])

    for k in range(_NBUF):
        start(k, k)

    def body(t, carry):
        j = t * _NBUF
        for k in range(_NBUF):
            wait(k)
            store(j + k, k)

            @pl.when(j + k + _NBUF < _NCH)
            def _():
                start(j + k + _NBUF, k)

        return carry

    lax.fori_loop(0, _NCH // _NBUF, body, 0)


def _gather_fn(pt, idx):
    scratch = [pltpu.VMEM((_NCH, _CHUNK), jnp.int32)]
    scratch += [pltpu.VMEM((_CHUNK, 2 * D), jnp.float32) for _ in range(_NBUF)]
    scratch += [pltpu.SemaphoreType.DMA for _ in range(_NBUF)]

    def body(pt_ref, idx_ref, out_ref, idxv, b0, b1, b2, b3, s0, s1, s2, s3):
        _gather_body(pt_ref, idx_ref, out_ref, idxv,
                     (b0, b1, b2, b3), (s0, s1, s2, s3))

    return pl.kernel(
        body,
        mesh=plsc.VectorSubcoreMesh(core_axis_name="c", subcore_axis_name="s"),
        out_type=jax.ShapeDtypeStruct((2 * N, 2 * D), jnp.float32),
        scratch_types=scratch,
        compiler_params=pltpu.CompilerParams(use_tc_tiling_on_sc=True),
    )(pt, idx)


_VB = 16384


def _premul_body(ctt, ett, clkw, expw, rtab, tsb, clkb, expb, out):
    w_clk = clkw[...]
    w_exp = expw[...]
    tsb_v = tsb[...]
    rt = rtab[...]
    oh2 = (lax.broadcasted_iota(jnp.int32, (1, 8), 1) == 2).astype(jnp.float32)
    r2 = jnp.dot(oh2, rt, preferred_element_type=jnp.float32)
    cc = (jnp.dot(tsb_v, w_clk[D:2 * D], preferred_element_type=jnp.float32)
          + jnp.dot(r2, w_clk[2 * D:], preferred_element_type=jnp.float32)
          + clkb[...])
    ce = (jnp.dot(tsb_v, w_exp[D:2 * D], preferred_element_type=jnp.float32)
          + expb[...])
    dn = (((0,), (0,)), ((), ()))
    pc = lax.dot_general(ctt[...].astype(jnp.bfloat16),
                         w_clk[:D].astype(jnp.bfloat16), dn,
                         preferred_element_type=jnp.float32)
    pe = lax.dot_general(ett[...].astype(jnp.bfloat16),
                         w_exp[:D].astype(jnp.bfloat16), dn,
                         preferred_element_type=jnp.float32)
    out[...] = jnp.concatenate([pc + cc, pe + ce], axis=1)


_premul = pl.pallas_call(
    _premul_body,
    grid=(pl.cdiv(V, _VB),),
    in_specs=[
        pl.BlockSpec((D, _VB), lambda i: (0, i)),
        pl.BlockSpec((D, _VB), lambda i: (0, i)),
        pl.BlockSpec((3 * D, D), lambda i: (0, 0)),
        pl.BlockSpec((3 * D, D), lambda i: (0, 0)),
        pl.BlockSpec((8, D), lambda i: (0, 0)),
        pl.BlockSpec((1, D), lambda i: (0, 0)),
        pl.BlockSpec((1, D), lambda i: (0, 0)),
        pl.BlockSpec((1, D), lambda i: (0, 0)),
    ],
    out_specs=pl.BlockSpec((_VB, 2 * D), lambda i: (i, 0)),
    out_shape=jax.ShapeDtypeStruct((V, 2 * D), jnp.float32),
)


_BB = 16
_BLK = _BB * S
_GRID = B // _BB


def _fused_body(g1, g2, it, r1, st, ids, tsw, rtab, expw, clkw, o1, o2):
    w_clk = clkw[...]
    w_exp = expw[...]
    tsw_v = tsw[...]
    rt = rtab[...]

    uc = jnp.dot(tsw_v, w_clk[D:2 * D], preferred_element_type=jnp.float32)
    ue = jnp.dot(tsw_v, w_exp[D:2 * D], preferred_element_type=jnp.float32)
    rt6 = jnp.dot(rt, w_exp[2 * D:], preferred_element_type=jnp.float32)

    itv = it[...]
    l1 = jnp.log(itv - r1[...] + 1.0)
    l2 = jnp.log(itv - st[...] + 1.0)
    L1 = jnp.broadcast_to(l1[:, :, None], (_BB, S, D)).reshape(_BLK, D)
    L2 = jnp.broadcast_to(l2[:, :, None], (_BB, S, D)).reshape(_BLK, D)
    oh = (ids[...][:, :, None]
          == lax.broadcasted_iota(jnp.int32, (_BB, S, 8), 2)
          ).astype(jnp.float32).reshape(_BLK, 8)

    out1 = g1[...][:, :D] + L1 * uc
    out2 = (g2[...][:, D:] + L2 * ue
            + jnp.dot(oh, rt6, preferred_element_type=jnp.float32))
    o1[...] = out1.reshape(_BB, S, D)
    o2[...] = out2.reshape(_BB, S, D)


_fused = pl.pallas_call(
    _fused_body,
    grid=(_GRID,),
    in_specs=[
        pl.BlockSpec((_BLK, 2 * D), lambda i: (i, 0)),
        pl.BlockSpec((_BLK, 2 * D), lambda i: (i + _GRID, 0)),
        pl.BlockSpec((_BB, 1), lambda i: (i, 0)),
        pl.BlockSpec((_BB, S), lambda i: (i, 0)),
        pl.BlockSpec((_BB, S), lambda i: (i, 0)),
        pl.BlockSpec((_BB, S), lambda i: (i, 0)),
        pl.BlockSpec((1, D), lambda i: (0, 0)),
        pl.BlockSpec((8, D), lambda i: (0, 0)),
        pl.BlockSpec((3 * D, D), lambda i: (0, 0)),
        pl.BlockSpec((3 * D, D), lambda i: (0, 0)),
    ],
    out_specs=[
        pl.BlockSpec((_BB, S, D), lambda i: (i, 0, 0)),
        pl.BlockSpec((_BB, S, D), lambda i: (i, 0, 0)),
    ],
    out_shape=[
        jax.ShapeDtypeStruct((B, H, D), jnp.float32),
        jax.ShapeDtypeStruct((B, S, D), jnp.float32),
    ],
)


def kernel(row0, row1, row2, row3, row4, row5, row6, row7, click_table,
           exposure_table, rating_table, ts_w, ts_b, exp_w, exp_b, clk_w,
           clk_b):
    item_time = row6[:, -1]
    seq_items = row4[:, :-1]
    seq_ratings = row5[:, :-1]
    seq_times = row6[:, :-1]

    rt8 = jnp.zeros((8, D), jnp.float32).at[:R].set(rating_table)
    pt = _premul(click_table.T, exposure_table.T, clk_w, exp_w, rt8,
                 ts_b.reshape(1, D), clk_b.reshape(1, D),
                 exp_b.reshape(1, D))
    idx = jnp.concatenate(
        [row0.astype(jnp.int32).reshape(-1),
         seq_items.astype(jnp.int32).reshape(-1)]
    ).reshape(_NW, _NCH, _CHUNK)
    g = _gather_fn(pt, idx)

    o1, o2 = _fused(g, g, item_time.reshape(B, 1), row1, seq_times,
                    seq_ratings.astype(jnp.int32), ts_w, rt8, exp_w, clk_w)
    return o1, o2
